# baseline (speedup 1.0000x reference)
import functools

import jax
import jax.numpy as jnp
from jax import lax
from jax.experimental import pallas as pl
from jax.experimental.pallas import tpu as pltpu
from jax.experimental.pallas import tpu_sc as plsc

VOCAB_N = 1000000
HID = 64
BATCH_N = 4096
SEQ_N = 200

_PAIR = 2
_BP = BATCH_N // _PAIR
_HP = _PAIR * HID
_GP = 4 * _HP

_NC = 2
_NS = 16
_NW = _NC * _NS

_ROWS = SEQ_N * BATCH_N
_PER_W = _ROWS // _NW
_CH = 1024


def _sc_gather_body(idx_hbm, emb_hbm, out_hbm, idx_v, rows_v, sem):
    wid = lax.axis_index("s") * _NC + lax.axis_index("c")
    base = wid * _PER_W

    def chunk(j, carry):
        off = base + j * _CH
        pltpu.sync_copy(idx_hbm.at[pl.ds(off, _CH)], idx_v)
        pltpu.async_copy(emb_hbm.at[idx_v], rows_v, sem).wait()
        pltpu.sync_copy(rows_v, out_hbm.at[pl.ds(off, _CH)])
        return carry

    lax.fori_loop(0, _PER_W // _CH, chunk, 0)


@functools.cache
def _sc_gather():
    return functools.partial(
        pl.kernel,
        mesh=plsc.VectorSubcoreMesh(core_axis_name="c", subcore_axis_name="s"),
        compiler_params=pltpu.CompilerParams(use_tc_tiling_on_sc=False),
        out_type=jax.ShapeDtypeStruct((_ROWS, HID), jnp.bfloat16),
        scratch_types=[
            pltpu.VMEM((_CH,), jnp.int32),
            pltpu.VMEM((_CH, HID), jnp.bfloat16),
            pltpu.SemaphoreType.DMA,
        ],
    )(_sc_gather_body)


def _lstm_body(e_ref, w_ref, u_ref, b_ref, h_out, c_out, h_s, c_s):
    t = pl.program_id(0)

    @pl.when(t == 0)
    def _init():
        h_s[...] = jnp.zeros_like(h_s)
        c_s[...] = jnp.zeros_like(c_s)

    xt = e_ref[...]
    h = h_s[...]
    c = c_s[...]
    z = (jnp.dot(xt, w_ref[...], preferred_element_type=jnp.float32)
         + jnp.dot(h.astype(jnp.bfloat16), u_ref[...],
                   preferred_element_type=jnp.float32)
         + b_ref[...])
    gi = jax.nn.sigmoid(z[:, :_HP])
    gf = jax.nn.sigmoid(z[:, _HP:2 * _HP])
    gg = jnp.tanh(z[:, 2 * _HP:3 * _HP])
    go = jax.nn.sigmoid(z[:, 3 * _HP:])
    c_new = gf * c + gi * gg
    h_new = go * jnp.tanh(c_new)
    h_s[...] = h_new
    c_s[...] = c_new

    @pl.when(t == SEQ_N - 1)
    def _fin():
        h_out[...] = h_new
        c_out[...] = c_new


_lstm_call = pl.pallas_call(
    _lstm_body,
    grid=(SEQ_N,),
    in_specs=[
        pl.BlockSpec((_BP, _HP), lambda t: (t, 0)),
        pl.BlockSpec((_HP, _GP), lambda t: (0, 0)),
        pl.BlockSpec((_HP, _GP), lambda t: (0, 0)),
        pl.BlockSpec((1, _GP), lambda t: (0, 0)),
    ],
    out_specs=[
        pl.BlockSpec((_BP, _HP), lambda t: (0, 0)),
        pl.BlockSpec((_BP, _HP), lambda t: (0, 0)),
    ],
    out_shape=[
        jax.ShapeDtypeStruct((_BP, _HP), jnp.float32),
        jax.ShapeDtypeStruct((_BP, _HP), jnp.float32),
    ],
    scratch_shapes=[
        pltpu.VMEM((_BP, _HP), jnp.float32),
        pltpu.VMEM((_BP, _HP), jnp.float32),
    ],
)


def _pair_weights(W):
    eye = jnp.eye(_PAIR, dtype=W.dtype)
    return jnp.concatenate(
        [jnp.kron(eye, W[:, k * HID:(k + 1) * HID]) for k in range(4)], axis=1)


def kernel(x, emb, W, U, b):
    idx = jnp.swapaxes(x, 0, 1).reshape(-1)
    e = _sc_gather()(idx, emb.astype(jnp.bfloat16))
    e2 = e.reshape(SEQ_N * _BP, _HP)
    wbig = _pair_weights(W).astype(jnp.bfloat16)
    ubig = _pair_weights(U).astype(jnp.bfloat16)
    b2 = jnp.concatenate(
        [jnp.tile(b[k * HID:(k + 1) * HID], _PAIR) for k in range(4)])
    h2, c2 = _lstm_call(e2, wbig, ubig, b2.reshape(1, _GP))
    h = h2.reshape(BATCH_N, HID)
    c = c2.reshape(BATCH_N, HID)
    return (h, h, c)


# speedup vs baseline: 1.6275x; 1.6275x over previous
import functools

import jax
import jax.numpy as jnp
from jax import lax
from jax.experimental import pallas as pl
from jax.experimental.pallas import tpu as pltpu
from jax.experimental.pallas import tpu_sc as plsc

VOCAB_N = 1000000
HID = 64
BATCH_N = 4096
SEQ_N = 200

_PAIR = 2
_BP = BATCH_N // _PAIR
_HP = _PAIR * HID
_GP = 4 * _HP

_NC = 2
_NS = 16
_NW = _NC * _NS

_NCHK = 4
_SC_N = SEQ_N // _NCHK
_ROWS = _SC_N * BATCH_N
_PER_W = _ROWS // _NW
_CH = 800


def _sc_gather_body(idx_hbm, emb_hbm, out_hbm, idx_v, rows_v, sem):
    wid = lax.axis_index("s") * _NC + lax.axis_index("c")
    base = wid * _PER_W

    def chunk(j, carry):
        off = base + j * _CH
        pltpu.sync_copy(idx_hbm.at[pl.ds(off, _CH)], idx_v)
        pltpu.async_copy(emb_hbm.at[idx_v], rows_v, sem).wait()
        pltpu.sync_copy(rows_v, out_hbm.at[pl.ds(off, _CH)])
        return carry

    lax.fori_loop(0, _PER_W // _CH, chunk, 0)


@functools.cache
def _sc_gather():
    return functools.partial(
        pl.kernel,
        mesh=plsc.VectorSubcoreMesh(core_axis_name="c", subcore_axis_name="s"),
        compiler_params=pltpu.CompilerParams(use_tc_tiling_on_sc=False),
        out_type=jax.ShapeDtypeStruct((_ROWS, HID), jnp.float32),
        scratch_types=[
            pltpu.VMEM((_CH,), jnp.int32),
            pltpu.VMEM((_CH, HID), jnp.float32),
            pltpu.SemaphoreType.DMA,
        ],
    )(_sc_gather_body)


def _lstm_body(e_ref, w_ref, u_ref, b_ref, h0_ref, c0_ref, h_out, c_out,
               h_s, c_s):
    t = pl.program_id(0)

    @pl.when(t == 0)
    def _init():
        h_s[...] = h0_ref[...]
        c_s[...] = c0_ref[...]

    xt = e_ref[...].astype(jnp.bfloat16)
    h = h_s[...]
    c = c_s[...]
    z = (jnp.dot(xt, w_ref[...], preferred_element_type=jnp.float32)
         + jnp.dot(h.astype(jnp.bfloat16), u_ref[...],
                   preferred_element_type=jnp.float32)
         + b_ref[...])
    gi = jax.nn.sigmoid(z[:, :_HP])
    gf = jax.nn.sigmoid(z[:, _HP:2 * _HP])
    gg = jnp.tanh(z[:, 2 * _HP:3 * _HP])
    go = jax.nn.sigmoid(z[:, 3 * _HP:])
    c_new = gf * c + gi * gg
    h_new = go * jnp.tanh(c_new)
    h_s[...] = h_new
    c_s[...] = c_new

    @pl.when(t == _SC_N - 1)
    def _fin():
        h_out[...] = h_new
        c_out[...] = c_new


_lstm_call = pl.pallas_call(
    _lstm_body,
    grid=(_SC_N,),
    in_specs=[
        pl.BlockSpec((_BP, _HP), lambda t: (t, 0)),
        pl.BlockSpec((_HP, _GP), lambda t: (0, 0)),
        pl.BlockSpec((_HP, _GP), lambda t: (0, 0)),
        pl.BlockSpec((1, _GP), lambda t: (0, 0)),
        pl.BlockSpec((_BP, _HP), lambda t: (0, 0)),
        pl.BlockSpec((_BP, _HP), lambda t: (0, 0)),
    ],
    out_specs=[
        pl.BlockSpec((_BP, _HP), lambda t: (0, 0)),
        pl.BlockSpec((_BP, _HP), lambda t: (0, 0)),
    ],
    out_shape=[
        jax.ShapeDtypeStruct((_BP, _HP), jnp.float32),
        jax.ShapeDtypeStruct((_BP, _HP), jnp.float32),
    ],
    scratch_shapes=[
        pltpu.VMEM((_BP, _HP), jnp.float32),
        pltpu.VMEM((_BP, _HP), jnp.float32),
    ],
)


def _pair_weights(W):
    eye = jnp.eye(_PAIR, dtype=W.dtype)
    return jnp.concatenate(
        [jnp.kron(eye, W[:, k * HID:(k + 1) * HID]) for k in range(4)], axis=1)


def kernel(x, emb, W, U, b):
    idx = jnp.swapaxes(x, 0, 1).reshape(_NCHK, _ROWS)
    wbig = _pair_weights(W).astype(jnp.bfloat16)
    ubig = _pair_weights(U).astype(jnp.bfloat16)
    b2 = jnp.concatenate(
        [jnp.tile(b[k * HID:(k + 1) * HID], _PAIR) for k in range(4)])
    b2 = b2.reshape(1, _GP)
    gather = _sc_gather()
    h2 = jnp.zeros((_BP, _HP), jnp.float32)
    c2 = jnp.zeros((_BP, _HP), jnp.float32)
    es = [gather(idx[k], emb) for k in range(_NCHK)]
    for k in range(_NCHK):
        e2 = es[k].reshape(_SC_N * _BP, _HP)
        h2, c2 = _lstm_call(e2, wbig, ubig, b2, h2, c2)
    h = h2.reshape(BATCH_N, HID)
    c = c2.reshape(BATCH_N, HID)
    return (h, h, c)


# speedup vs baseline: 1.6834x; 1.0343x over previous
import functools

import jax
import jax.numpy as jnp
from jax import lax
from jax.experimental import pallas as pl
from jax.experimental.pallas import tpu as pltpu
from jax.experimental.pallas import tpu_sc as plsc

VOCAB_N = 1000000
HID = 64
BATCH_N = 4096
SEQ_N = 200

_PAIR = 2
_BP = BATCH_N // _PAIR
_HP = _PAIR * HID
_GP = 4 * _HP

_NC = 2
_NS = 16
_NW = _NC * _NS

_NCHK = 4
_SC_N = SEQ_N // _NCHK
_ROWS = _SC_N * BATCH_N
_PER_W = _ROWS // _NW
_CH = 800


def _sc_gather_body(idx_hbm, emb_hbm, out_hbm, idx_v, rows_v, sem):
    wid = lax.axis_index("s") * _NC + lax.axis_index("c")
    base = wid * _PER_W

    def chunk(j, carry):
        off = base + j * _CH
        pltpu.sync_copy(idx_hbm.at[pl.ds(off, _CH)], idx_v)
        pltpu.async_copy(emb_hbm.at[idx_v], rows_v, sem).wait()
        pltpu.sync_copy(rows_v, out_hbm.at[pl.ds(off, _CH)])
        return carry

    lax.fori_loop(0, _PER_W // _CH, chunk, 0)


@functools.cache
def _sc_gather():
    return functools.partial(
        pl.kernel,
        mesh=plsc.VectorSubcoreMesh(core_axis_name="c", subcore_axis_name="s"),
        compiler_params=pltpu.CompilerParams(use_tc_tiling_on_sc=False),
        out_type=jax.ShapeDtypeStruct((_ROWS, HID), jnp.float32),
        scratch_types=[
            pltpu.VMEM((_CH,), jnp.int32),
            pltpu.VMEM((_CH, HID), jnp.float32),
            pltpu.SemaphoreType.DMA,
        ],
    )(_sc_gather_body)


def _lstm_body(e_ref, w_ref, u_ref, b_ref, h0_ref, c0_ref, h_out, c_out,
               h_s, c_s):
    t = pl.program_id(0)

    @pl.when(t == 0)
    def _init():
        h_s[...] = h0_ref[...]
        c_s[...] = c0_ref[...]

    xt = e_ref[...].astype(jnp.bfloat16)
    h = h_s[...]
    c = c_s[...]
    z = (jnp.dot(xt, w_ref[...], preferred_element_type=jnp.float32)
         + jnp.dot(h.astype(jnp.bfloat16), u_ref[...],
                   preferred_element_type=jnp.float32)
         + b_ref[...])
    def _sig(v):
        return 0.5 * jnp.tanh(0.5 * v) + 0.5

    gi = _sig(z[:, :_HP])
    gf = _sig(z[:, _HP:2 * _HP])
    gg = jnp.tanh(z[:, 2 * _HP:3 * _HP])
    go = _sig(z[:, 3 * _HP:])
    c_new = gf * c + gi * gg
    h_new = go * jnp.tanh(c_new)
    h_s[...] = h_new
    c_s[...] = c_new

    @pl.when(t == _SC_N - 1)
    def _fin():
        h_out[...] = h_new
        c_out[...] = c_new


_lstm_call = pl.pallas_call(
    _lstm_body,
    grid=(_SC_N,),
    in_specs=[
        pl.BlockSpec((_BP, _HP), lambda t: (t, 0)),
        pl.BlockSpec((_HP, _GP), lambda t: (0, 0)),
        pl.BlockSpec((_HP, _GP), lambda t: (0, 0)),
        pl.BlockSpec((1, _GP), lambda t: (0, 0)),
        pl.BlockSpec((_BP, _HP), lambda t: (0, 0)),
        pl.BlockSpec((_BP, _HP), lambda t: (0, 0)),
    ],
    out_specs=[
        pl.BlockSpec((_BP, _HP), lambda t: (0, 0)),
        pl.BlockSpec((_BP, _HP), lambda t: (0, 0)),
    ],
    out_shape=[
        jax.ShapeDtypeStruct((_BP, _HP), jnp.float32),
        jax.ShapeDtypeStruct((_BP, _HP), jnp.float32),
    ],
    scratch_shapes=[
        pltpu.VMEM((_BP, _HP), jnp.float32),
        pltpu.VMEM((_BP, _HP), jnp.float32),
    ],
)


def _pair_weights(W):
    eye = jnp.eye(_PAIR, dtype=W.dtype)
    return jnp.concatenate(
        [jnp.kron(eye, W[:, k * HID:(k + 1) * HID]) for k in range(4)], axis=1)


def kernel(x, emb, W, U, b):
    idx = jnp.swapaxes(x, 0, 1).reshape(_NCHK, _ROWS)
    wbig = _pair_weights(W).astype(jnp.bfloat16)
    ubig = _pair_weights(U).astype(jnp.bfloat16)
    b2 = jnp.concatenate(
        [jnp.tile(b[k * HID:(k + 1) * HID], _PAIR) for k in range(4)])
    b2 = b2.reshape(1, _GP)
    gather = _sc_gather()
    h2 = jnp.zeros((_BP, _HP), jnp.float32)
    c2 = jnp.zeros((_BP, _HP), jnp.float32)
    es = [gather(idx[k], emb) for k in range(_NCHK)]
    for k in range(_NCHK):
        e2 = es[k].reshape(_SC_N * _BP, _HP)
        h2, c2 = _lstm_call(e2, wbig, ubig, b2, h2, c2)
    h = h2.reshape(BATCH_N, HID)
    c = c2.reshape(BATCH_N, HID)
    return (h, h, c)


# speedup vs baseline: 1.6864x; 1.0018x over previous
import functools

import jax
import jax.numpy as jnp
from jax import lax
from jax.experimental import pallas as pl
from jax.experimental.pallas import tpu as pltpu
from jax.experimental.pallas import tpu_sc as plsc

VOCAB_N = 1000000
HID = 64
BATCH_N = 4096
SEQ_N = 200

_PAIR = 2
_BP = BATCH_N // _PAIR
_HP = _PAIR * HID
_GP = 4 * _HP

_NC = 2
_NS = 16
_NW = _NC * _NS

_NCHK = 8
_SC_N = SEQ_N // _NCHK
_ROWS = _SC_N * BATCH_N
_PER_W = _ROWS // _NW
_CH = 800


def _sc_gather_body(idx_hbm, emb_hbm, out_hbm, idx_v, rows_v, sem):
    wid = lax.axis_index("s") * _NC + lax.axis_index("c")
    base = wid * _PER_W

    def chunk(j, carry):
        off = base + j * _CH
        pltpu.sync_copy(idx_hbm.at[pl.ds(off, _CH)], idx_v)
        pltpu.async_copy(emb_hbm.at[idx_v], rows_v, sem).wait()
        pltpu.sync_copy(rows_v, out_hbm.at[pl.ds(off, _CH)])
        return carry

    lax.fori_loop(0, _PER_W // _CH, chunk, 0)


@functools.cache
def _sc_gather():
    return functools.partial(
        pl.kernel,
        mesh=plsc.VectorSubcoreMesh(core_axis_name="c", subcore_axis_name="s"),
        compiler_params=pltpu.CompilerParams(use_tc_tiling_on_sc=False),
        out_type=jax.ShapeDtypeStruct((_ROWS, HID), jnp.float32),
        scratch_types=[
            pltpu.VMEM((_CH,), jnp.int32),
            pltpu.VMEM((_CH, HID), jnp.float32),
            pltpu.SemaphoreType.DMA,
        ],
    )(_sc_gather_body)


def _lstm_body(e_ref, w_ref, u_ref, b_ref, h0_ref, c0_ref, h_out, c_out,
               h_s, c_s):
    t = pl.program_id(0)

    @pl.when(t == 0)
    def _init():
        h_s[...] = h0_ref[...]
        c_s[...] = c0_ref[...]

    xt = e_ref[...].astype(jnp.bfloat16)
    h = h_s[...]
    c = c_s[...]
    z = (jnp.dot(xt, w_ref[...], preferred_element_type=jnp.float32)
         + jnp.dot(h.astype(jnp.bfloat16), u_ref[...],
                   preferred_element_type=jnp.float32)
         + b_ref[...])
    def _sig(v):
        return 0.5 * jnp.tanh(0.5 * v) + 0.5

    gi = _sig(z[:, :_HP])
    gf = _sig(z[:, _HP:2 * _HP])
    gg = jnp.tanh(z[:, 2 * _HP:3 * _HP])
    go = _sig(z[:, 3 * _HP:])
    c_new = gf * c + gi * gg
    h_new = go * jnp.tanh(c_new)
    h_s[...] = h_new
    c_s[...] = c_new

    @pl.when(t == _SC_N - 1)
    def _fin():
        h_out[...] = h_new
        c_out[...] = c_new


_lstm_call = pl.pallas_call(
    _lstm_body,
    grid=(_SC_N,),
    in_specs=[
        pl.BlockSpec((_BP, _HP), lambda t: (t, 0)),
        pl.BlockSpec((_HP, _GP), lambda t: (0, 0)),
        pl.BlockSpec((_HP, _GP), lambda t: (0, 0)),
        pl.BlockSpec((1, _GP), lambda t: (0, 0)),
        pl.BlockSpec((_BP, _HP), lambda t: (0, 0)),
        pl.BlockSpec((_BP, _HP), lambda t: (0, 0)),
    ],
    out_specs=[
        pl.BlockSpec((_BP, _HP), lambda t: (0, 0)),
        pl.BlockSpec((_BP, _HP), lambda t: (0, 0)),
    ],
    out_shape=[
        jax.ShapeDtypeStruct((_BP, _HP), jnp.float32),
        jax.ShapeDtypeStruct((_BP, _HP), jnp.float32),
    ],
    scratch_shapes=[
        pltpu.VMEM((_BP, _HP), jnp.float32),
        pltpu.VMEM((_BP, _HP), jnp.float32),
    ],
)


def _pair_weights(W):
    eye = jnp.eye(_PAIR, dtype=W.dtype)
    return jnp.concatenate(
        [jnp.kron(eye, W[:, k * HID:(k + 1) * HID]) for k in range(4)], axis=1)


def kernel(x, emb, W, U, b):
    idx = jnp.swapaxes(x, 0, 1).reshape(_NCHK, _ROWS)
    wbig = _pair_weights(W).astype(jnp.bfloat16)
    ubig = _pair_weights(U).astype(jnp.bfloat16)
    b2 = jnp.concatenate(
        [jnp.tile(b[k * HID:(k + 1) * HID], _PAIR) for k in range(4)])
    b2 = b2.reshape(1, _GP)
    gather = _sc_gather()
    h2 = jnp.zeros((_BP, _HP), jnp.float32)
    c2 = jnp.zeros((_BP, _HP), jnp.float32)
    es = [gather(idx[k], emb) for k in range(_NCHK)]
    for k in range(_NCHK):
        e2 = es[k].reshape(_SC_N * _BP, _HP)
        h2, c2 = _lstm_call(e2, wbig, ubig, b2, h2, c2)
    h = h2.reshape(BATCH_N, HID)
    c = c2.reshape(BATCH_N, HID)
    return (h, h, c)


# speedup vs baseline: 1.8176x; 1.0778x over previous
import functools

import jax
import jax.numpy as jnp
from jax import lax
from jax.experimental import pallas as pl
from jax.experimental.pallas import tpu as pltpu
from jax.experimental.pallas import tpu_sc as plsc

VOCAB_N = 1000000
HID = 64
BATCH_N = 4096
SEQ_N = 200

_PAIR = 2
_BP = BATCH_N // _PAIR
_HP = _PAIR * HID
_GP = 4 * _HP

_NC = 2
_NS = 16
_NW = _NC * _NS

_NCHK = 8
_SC_N = SEQ_N // _NCHK
_ROWS = _SC_N * BATCH_N
_PER_W = _ROWS // _NW
_CH = 800


def _sc_gather_body(idx_hbm, emb_hbm, out_hbm, idx_v, rows_v, sem):
    wid = lax.axis_index("s") * _NC + lax.axis_index("c")
    base = wid * _PER_W

    def chunk(j, carry):
        off = base + j * _CH
        pltpu.sync_copy(idx_hbm.at[pl.ds(off, _CH)], idx_v)
        pltpu.async_copy(emb_hbm.at[idx_v], rows_v, sem).wait()
        pltpu.sync_copy(rows_v, out_hbm.at[pl.ds(off, _CH)])
        return carry

    lax.fori_loop(0, _PER_W // _CH, chunk, 0)


@functools.cache
def _sc_gather():
    return functools.partial(
        pl.kernel,
        mesh=plsc.VectorSubcoreMesh(core_axis_name="c", subcore_axis_name="s"),
        compiler_params=pltpu.CompilerParams(use_tc_tiling_on_sc=False),
        out_type=jax.ShapeDtypeStruct((_ROWS, HID), jnp.float32),
        scratch_types=[
            pltpu.VMEM((_CH,), jnp.int32),
            pltpu.VMEM((_CH, HID), jnp.float32),
            pltpu.SemaphoreType.DMA,
        ],
    )(_sc_gather_body)


def _lstm_body(e_ref, wu_ref, b_ref, h0_ref, c0_ref, h_out, c_out,
               h_s, c_s):
    t = pl.program_id(0)

    @pl.when(t == 0)
    def _init():
        h_s[...] = h0_ref[...]
        c_s[...] = c0_ref[...]

    xt = e_ref[...].astype(jnp.bfloat16)
    h = h_s[...]
    c = c_s[...]
    xh = jnp.concatenate([xt, h.astype(jnp.bfloat16)], axis=1)
    z = (jnp.dot(xh, wu_ref[...], preferred_element_type=jnp.float32)
         + b_ref[...])
    def _sig(v):
        return 0.5 * jnp.tanh(0.5 * v) + 0.5

    gi = _sig(z[:, :_HP])
    gf = _sig(z[:, _HP:2 * _HP])
    gg = jnp.tanh(z[:, 2 * _HP:3 * _HP])
    go = _sig(z[:, 3 * _HP:])
    c_new = gf * c + gi * gg
    h_new = go * jnp.tanh(c_new)
    h_s[...] = h_new
    c_s[...] = c_new

    @pl.when(t == _SC_N - 1)
    def _fin():
        h_out[...] = h_new
        c_out[...] = c_new


_lstm_call = pl.pallas_call(
    _lstm_body,
    grid=(_SC_N,),
    in_specs=[
        pl.BlockSpec((_BP, _HP), lambda t: (t, 0)),
        pl.BlockSpec((2 * _HP, _GP), lambda t: (0, 0)),
        pl.BlockSpec((1, _GP), lambda t: (0, 0)),
        pl.BlockSpec((_BP, _HP), lambda t: (0, 0)),
        pl.BlockSpec((_BP, _HP), lambda t: (0, 0)),
    ],
    out_specs=[
        pl.BlockSpec((_BP, _HP), lambda t: (0, 0)),
        pl.BlockSpec((_BP, _HP), lambda t: (0, 0)),
    ],
    out_shape=[
        jax.ShapeDtypeStruct((_BP, _HP), jnp.float32),
        jax.ShapeDtypeStruct((_BP, _HP), jnp.float32),
    ],
    scratch_shapes=[
        pltpu.VMEM((_BP, _HP), jnp.float32),
        pltpu.VMEM((_BP, _HP), jnp.float32),
    ],
)


def _pair_weights(W):
    eye = jnp.eye(_PAIR, dtype=W.dtype)
    return jnp.concatenate(
        [jnp.kron(eye, W[:, k * HID:(k + 1) * HID]) for k in range(4)], axis=1)


def kernel(x, emb, W, U, b):
    idx = jnp.swapaxes(x, 0, 1).reshape(_NCHK, _ROWS)
    wu = jnp.concatenate(
        [_pair_weights(W), _pair_weights(U)], axis=0).astype(jnp.bfloat16)
    b2 = jnp.concatenate(
        [jnp.tile(b[k * HID:(k + 1) * HID], _PAIR) for k in range(4)])
    b2 = b2.reshape(1, _GP)
    gather = _sc_gather()
    h2 = jnp.zeros((_BP, _HP), jnp.float32)
    c2 = jnp.zeros((_BP, _HP), jnp.float32)
    es = [gather(idx[k], emb) for k in range(_NCHK)]
    for k in range(_NCHK):
        e2 = es[k].reshape(_SC_N * _BP, _HP)
        h2, c2 = _lstm_call(e2, wu, b2, h2, c2)
    h = h2.reshape(BATCH_N, HID)
    c = c2.reshape(BATCH_N, HID)
    return (h, h, c)
